# Initial kernel scaffold; baseline (speedup 1.0000x reference)
import jax
import jax.numpy as jnp
from jax.experimental import pallas as pl


def kernel(edge_index, edge_values, user_emb, item_emb):
    raise NotImplementedError("write your pallas kernel here")


# speedup vs baseline: 4.2519x; 4.2519x over previous
import dataclasses
import functools

import jax
import jax.numpy as jnp
from jax import lax
from jax.experimental import pallas as pl
from jax.experimental.pallas import tpu as pltpu
from jax.experimental.pallas import tpu_sc as plsc

N_USERS = 5000
N_ITEMS = 5000
N_NODES = N_USERS + N_ITEMS
EMBED = 128
HALF = EMBED // 2
N_LAYERS = 3

N_CORES = 2
N_SUBCORES = 16
LANES = 16
CHUNK = 128
GROUP = 32
N_PAD = 10240
ROWS_PER_TILE = N_PAD // N_SUBCORES
ROW_CHUNK = 64
N_ROW_CHUNKS = ROWS_PER_TILE // ROW_CHUNK


def _lightgcn_sc(emb2, srcs, dsts, ws):
    n_chunks = srcs.shape[1]

    mesh = plsc.VectorSubcoreMesh(
        core_axis_name="core", subcore_axis_name="subcore")

    cp = pltpu.CompilerParams()
    for fld, val in (("needs_layout_passes", False),
                     ("use_tc_tiling_on_sc", False)):
        if fld in pltpu.CompilerParams.__dataclass_fields__:
            cp = dataclasses.replace(cp, **{fld: val})

    @functools.partial(
        pl.kernel,
        out_type=jax.ShapeDtypeStruct((N_CORES, N_PAD, HALF), jnp.float32),
        mesh=mesh,
        compiler_params=cp,
        scratch_types=[
            pltpu.VMEM_SHARED((N_PAD, HALF), jnp.float32),
            pltpu.VMEM_SHARED((N_PAD, HALF), jnp.float32),
            pltpu.VMEM((GROUP, CHUNK), jnp.int32),
            pltpu.VMEM((GROUP, CHUNK), jnp.int32),
            pltpu.VMEM((GROUP * CHUNK,), jnp.float32),
            pltpu.VMEM((CHUNK, HALF), jnp.float32),
            pltpu.VMEM((ROW_CHUNK, HALF), jnp.float32),
            pltpu.VMEM((ROW_CHUNK, HALF), jnp.float32),
            pltpu.VMEM((ROW_CHUNK, HALF), jnp.float32),
        ],
    )
    def k(emb_hbm, src_hbm, dst_hbm, w_hbm, out_hbm,
          tab_a, tab_b, src_v, dst_v, w_v, msg_v, ta, tb, tz):
        c = lax.axis_index("core")
        s = lax.axis_index("subcore")
        r0 = s * ROWS_PER_TILE

        zero16 = jnp.zeros((LANES,), jnp.float32)

        @pl.loop(0, ROW_CHUNK)
        def _(r):
            for v in range(HALF // LANES):
                tz[r, pl.ds(v * LANES, LANES)] = zero16

        for kk in range(N_ROW_CHUNKS):
            rows = pl.ds(r0 + kk * ROW_CHUNK, ROW_CHUNK)
            pltpu.sync_copy(emb_hbm.at[c, rows], ta)
            pltpu.sync_copy(ta, tab_a.at[rows])
            pltpu.sync_copy(ta, out_hbm.at[c, rows])
            pltpu.sync_copy(tz, tab_b.at[rows])
        plsc.subcore_barrier()

        def edge_pass(cur, nxt):
            @pl.loop(0, n_chunks // GROUP)
            def _(g):
                pltpu.sync_copy(src_hbm.at[s, pl.ds(g * GROUP, GROUP)], src_v)
                pltpu.sync_copy(dst_hbm.at[s, pl.ds(g * GROUP, GROUP)], dst_v)
                pltpu.sync_copy(
                    w_hbm.at[s, pl.ds(g * GROUP * CHUNK, GROUP * CHUNK)], w_v)

                @pl.loop(0, GROUP)
                def _(j):
                    pltpu.sync_copy(cur.at[src_v.at[j]], msg_v)

                    @pl.loop(0, CHUNK)
                    def _(e):
                        wv = plsc.load_gather(
                            w_v, [jnp.full((LANES,), j * CHUNK + e, jnp.int32)])
                        for v in range(HALF // LANES):
                            sl = pl.ds(v * LANES, LANES)
                            msg_v[e, sl] = msg_v[e, sl] * wv

                    pltpu.sync_copy(msg_v, nxt.at[dst_v.at[j]], add=True)

        def accum_out(nxt, scale=None):
            for kk in range(N_ROW_CHUNKS):
                rows = pl.ds(r0 + kk * ROW_CHUNK, ROW_CHUNK)
                pltpu.sync_copy(nxt.at[rows], ta)
                pltpu.sync_copy(out_hbm.at[c, rows], tb)

                @pl.loop(0, ROW_CHUNK)
                def _(r):
                    for v in range(HALF // LANES):
                        sl = pl.ds(v * LANES, LANES)
                        acc = tb[r, sl] + ta[r, sl]
                        if scale is not None:
                            acc = acc * scale
                        tb[r, sl] = acc

                pltpu.sync_copy(tb, out_hbm.at[c, rows])

        def zero_table(tab):
            for kk in range(N_ROW_CHUNKS):
                rows = pl.ds(r0 + kk * ROW_CHUNK, ROW_CHUNK)
                pltpu.sync_copy(tz, tab.at[rows])

        edge_pass(tab_a, tab_b)
        plsc.subcore_barrier()
        accum_out(tab_b)
        zero_table(tab_a)
        plsc.subcore_barrier()

        edge_pass(tab_b, tab_a)
        plsc.subcore_barrier()
        accum_out(tab_a)
        zero_table(tab_b)
        plsc.subcore_barrier()

        edge_pass(tab_a, tab_b)
        plsc.subcore_barrier()
        accum_out(tab_b, scale=0.25)

    return k(emb2, srcs, dsts, ws)


def kernel(edge_index, edge_values, user_emb, item_emb):
    n_edges = edge_values.shape[0]
    step = GROUP * CHUNK
    per_tile = -(-n_edges // (N_SUBCORES * step)) * step
    n_pad = N_SUBCORES * per_tile - n_edges

    dst = edge_index[0].astype(jnp.int32)
    src = edge_index[1].astype(jnp.int32)
    w = edge_values.astype(jnp.float32)
    if n_pad:
        zpad = jnp.zeros((n_pad,), jnp.int32)
        dst = jnp.concatenate([dst, zpad])
        src = jnp.concatenate([src, zpad])
        w = jnp.concatenate([w, jnp.zeros((n_pad,), jnp.float32)])

    srcs = src.reshape(N_SUBCORES, per_tile // CHUNK, CHUNK)
    dsts = dst.reshape(N_SUBCORES, per_tile // CHUNK, CHUNK)
    ws = w.reshape(N_SUBCORES, per_tile)

    all_emb = jnp.concatenate([
        user_emb, item_emb,
        jnp.zeros((N_PAD - N_NODES, EMBED), jnp.float32)], axis=0)
    emb2 = all_emb.reshape(N_PAD, N_CORES, HALF).transpose(1, 0, 2)

    out = _lightgcn_sc(emb2, srcs, dsts, ws)
    res = out.transpose(1, 0, 2).reshape(N_PAD, EMBED)
    return (res[:N_USERS], res[N_USERS:N_NODES])


# speedup vs baseline: 5.7867x; 1.3610x over previous
import dataclasses
import functools

import jax
import jax.numpy as jnp
from jax import lax
from jax.experimental import pallas as pl
from jax.experimental.pallas import tpu as pltpu
from jax.experimental.pallas import tpu_sc as plsc

N_USERS = 5000
N_ITEMS = 5000
N_NODES = N_USERS + N_ITEMS
EMBED = 128
HALF = EMBED // 2
N_LAYERS = 3

N_CORES = 2
N_SUBCORES = 16
LANES = 16
CHUNK = 128
GROUP = 32
N_PAD = 10240
ROWS_PER_TILE = N_PAD // N_SUBCORES
ROW_CHUNK = 64
N_ROW_CHUNKS = ROWS_PER_TILE // ROW_CHUNK


def _lightgcn_sc(emb2, srcs, dsts, ws):
    n_chunks = srcs.shape[1]

    mesh = plsc.VectorSubcoreMesh(
        core_axis_name="core", subcore_axis_name="subcore")

    cp = pltpu.CompilerParams()
    for fld, val in (("needs_layout_passes", False),
                     ("use_tc_tiling_on_sc", False)):
        if fld in pltpu.CompilerParams.__dataclass_fields__:
            cp = dataclasses.replace(cp, **{fld: val})

    @functools.partial(
        pl.kernel,
        out_type=jax.ShapeDtypeStruct((N_CORES, N_PAD, HALF), jnp.float32),
        mesh=mesh,
        compiler_params=cp,
        scratch_types=[
            pltpu.VMEM_SHARED((N_PAD, HALF), jnp.float32),
            pltpu.VMEM_SHARED((N_PAD, HALF), jnp.float32),
            pltpu.VMEM((GROUP, CHUNK), jnp.int32),
            pltpu.VMEM((GROUP, CHUNK), jnp.int32),
            pltpu.VMEM((GROUP * CHUNK,), jnp.float32),
            pltpu.VMEM((CHUNK, HALF), jnp.float32),
            pltpu.VMEM((CHUNK, HALF), jnp.float32),
            pltpu.VMEM((ROW_CHUNK, HALF), jnp.float32),
            pltpu.VMEM((ROW_CHUNK, HALF), jnp.float32),
            pltpu.VMEM((ROW_CHUNK, HALF), jnp.float32),
            pltpu.SemaphoreType.DMA,
            pltpu.SemaphoreType.DMA,
            pltpu.SemaphoreType.DMA,
            pltpu.SemaphoreType.DMA,
        ],
    )
    def k(emb_hbm, src_hbm, dst_hbm, w_hbm, out_hbm,
          tab_a, tab_b, src_v, dst_v, w_v, msg_a, msg_b, ta, tb, tz,
          gs_a, gs_b, ss_a, ss_b):
        c = lax.axis_index("core")
        s = lax.axis_index("subcore")
        r0 = s * ROWS_PER_TILE

        zero16 = jnp.zeros((LANES,), jnp.float32)

        @pl.loop(0, ROW_CHUNK)
        def _(r):
            for v in range(HALF // LANES):
                tz[r, pl.ds(v * LANES, LANES)] = zero16

        for kk in range(N_ROW_CHUNKS):
            rows = pl.ds(r0 + kk * ROW_CHUNK, ROW_CHUNK)
            pltpu.sync_copy(emb_hbm.at[c, rows], ta)
            pltpu.sync_copy(ta, tab_a.at[rows])
            pltpu.sync_copy(ta, out_hbm.at[c, rows])
            pltpu.sync_copy(tz, tab_b.at[rows])
        plsc.subcore_barrier()

        def edge_pass(cur, nxt):
            def scale(buf, j):
                @pl.loop(0, CHUNK, unroll=8)
                def _(e):
                    wv = plsc.load_gather(
                        w_v, [jnp.full((LANES,), j * CHUNK + e, jnp.int32)])
                    for v in range(HALF // LANES):
                        sl = pl.ds(v * LANES, LANES)
                        buf[e, sl] = buf[e, sl] * wv

            def start_gather(buf, sem, j):
                pltpu.async_copy(cur.at[src_v.at[j]], buf, sem)

            def wait_gather(buf, sem, j):
                pltpu.make_async_copy(cur.at[src_v.at[j]], buf, sem).wait()

            def start_scatter(buf, sem, j):
                pltpu.async_copy(buf, nxt.at[dst_v.at[j]], sem, add=True)

            def wait_scatter(buf, sem, j):
                pltpu.make_async_copy(
                    buf, nxt.at[dst_v.at[j]], sem).wait()

            @pl.loop(0, n_chunks // GROUP)
            def _(g):
                pltpu.sync_copy(src_hbm.at[s, pl.ds(g * GROUP, GROUP)], src_v)
                pltpu.sync_copy(dst_hbm.at[s, pl.ds(g * GROUP, GROUP)], dst_v)
                pltpu.sync_copy(
                    w_hbm.at[s, pl.ds(g * GROUP * CHUNK, GROUP * CHUNK)], w_v)

                start_gather(msg_a, gs_a, 0)
                start_gather(msg_b, gs_b, 1)

                @pl.loop(0, GROUP // 2)
                def _(p):
                    j0 = 2 * p
                    j1 = 2 * p + 1
                    wait_gather(msg_a, gs_a, j0)
                    scale(msg_a, j0)
                    start_scatter(msg_a, ss_a, j0)
                    wait_gather(msg_b, gs_b, j1)
                    scale(msg_b, j1)
                    start_scatter(msg_b, ss_b, j1)

                    @pl.when(p < GROUP // 2 - 1)
                    def _():
                        wait_scatter(msg_a, ss_a, j0)
                        start_gather(msg_a, gs_a, j0 + 2)
                        wait_scatter(msg_b, ss_b, j1)
                        start_gather(msg_b, gs_b, j1 + 2)

                wait_scatter(msg_a, ss_a, GROUP - 2)
                wait_scatter(msg_b, ss_b, GROUP - 1)

        def accum_out(nxt, scale=None):
            for kk in range(N_ROW_CHUNKS):
                rows = pl.ds(r0 + kk * ROW_CHUNK, ROW_CHUNK)
                pltpu.sync_copy(nxt.at[rows], ta)
                pltpu.sync_copy(out_hbm.at[c, rows], tb)

                @pl.loop(0, ROW_CHUNK)
                def _(r):
                    for v in range(HALF // LANES):
                        sl = pl.ds(v * LANES, LANES)
                        acc = tb[r, sl] + ta[r, sl]
                        if scale is not None:
                            acc = acc * scale
                        tb[r, sl] = acc

                pltpu.sync_copy(tb, out_hbm.at[c, rows])

        def zero_table(tab):
            for kk in range(N_ROW_CHUNKS):
                rows = pl.ds(r0 + kk * ROW_CHUNK, ROW_CHUNK)
                pltpu.sync_copy(tz, tab.at[rows])

        edge_pass(tab_a, tab_b)
        plsc.subcore_barrier()
        accum_out(tab_b)
        zero_table(tab_a)
        plsc.subcore_barrier()

        edge_pass(tab_b, tab_a)
        plsc.subcore_barrier()
        accum_out(tab_a)
        zero_table(tab_b)
        plsc.subcore_barrier()

        edge_pass(tab_a, tab_b)
        plsc.subcore_barrier()
        accum_out(tab_b, scale=0.25)

    return k(emb2, srcs, dsts, ws)


def kernel(edge_index, edge_values, user_emb, item_emb):
    n_edges = edge_values.shape[0]
    step = GROUP * CHUNK
    per_tile = -(-n_edges // (N_SUBCORES * step)) * step
    n_pad = N_SUBCORES * per_tile - n_edges

    dst = edge_index[0].astype(jnp.int32)
    src = edge_index[1].astype(jnp.int32)
    w = edge_values.astype(jnp.float32)
    if n_pad:
        zpad = jnp.zeros((n_pad,), jnp.int32)
        dst = jnp.concatenate([dst, zpad])
        src = jnp.concatenate([src, zpad])
        w = jnp.concatenate([w, jnp.zeros((n_pad,), jnp.float32)])

    srcs = src.reshape(N_SUBCORES, per_tile // CHUNK, CHUNK)
    dsts = dst.reshape(N_SUBCORES, per_tile // CHUNK, CHUNK)
    ws = w.reshape(N_SUBCORES, per_tile)

    all_emb = jnp.concatenate([
        user_emb, item_emb,
        jnp.zeros((N_PAD - N_NODES, EMBED), jnp.float32)], axis=0)
    emb2 = all_emb.reshape(N_PAD, N_CORES, HALF).transpose(1, 0, 2)

    out = _lightgcn_sc(emb2, srcs, dsts, ws)
    res = out.transpose(1, 0, 2).reshape(N_PAD, EMBED)
    return (res[:N_USERS], res[N_USERS:N_NODES])
